# Initial kernel scaffold; baseline (speedup 1.0000x reference)
import jax
import jax.numpy as jnp
from jax.experimental import pallas as pl


def kernel(node_feat, edge_index, edge_weight, all_embedding, wave_embedding, w_n2l_W, w_n2l_b, conv_W, conv_b, merge_W, merge_b):
    raise NotImplementedError("write your pallas kernel here")


# speedup vs baseline: 3.2139x; 3.2139x over previous
import functools

import jax
import jax.numpy as jnp
from jax import lax
from jax.experimental import pallas as pl
from jax.experimental.pallas import tpu as pltpu
from jax.experimental.pallas import tpu_sc as plsc

N = 10000
NP = 10240
L = 128
TWO_L = 256
FOUR_L = 512
T = 4
E = 80000
K = 128
NUM_CHUNKS = E // K
NW = 32
ROWS_PER_TILE = NP // 16
B = 2000
GRID = N // B


_mesh = plsc.VectorSubcoreMesh(core_axis_name="c", subcore_axis_name="s")


@functools.partial(
    pl.kernel,
    out_type=[jax.ShapeDtypeStruct((2, NP, L), jnp.float32) for _ in range(T)],
    mesh=_mesh,
    scratch_types=[
        pltpu.VMEM((K,), jnp.int32),
        pltpu.VMEM((K,), jnp.int32),
        pltpu.VMEM((K,), jnp.float32),
        pltpu.VMEM((K, L), jnp.float32),
        pltpu.VMEM((K, L), jnp.float32),
        pltpu.VMEM_SHARED((NP, L), jnp.float32),
        pltpu.SemaphoreType.DMA,
    ],
)
def _spmm_all_types(ch0, ch1, ch2, ch3, esrc, edst, ew, o0, o1, o2, o3,
                    sidx, didx, wv, rows, zbuf, acc, sem):
    cid = lax.axis_index("c")
    sid = lax.axis_index("s")
    wid = sid * 2 + cid
    row0 = sid * ROWS_PER_TILE

    z16 = jnp.zeros((16,), jnp.float32)

    def _zrow(r, carry):
        for c in range(8):
            zbuf[r, pl.ds(16 * c, 16)] = z16
        return carry

    lax.fori_loop(0, K, _zrow, 0)

    chs = [ch0, ch1, ch2, ch3]
    outs = [o0, o1, o2, o3]
    nchunks = (NUM_CHUNKS - 1 - wid) // NW + 1

    for t in range(T):
        for b5 in range(5):
            pltpu.sync_copy(zbuf, acc.at[pl.ds(row0 + K * b5, K)])
        plsc.subcore_barrier()

        def _chunk(i, carry, _t=t):
            base = _t * E + (wid + i * NW) * K
            pltpu.sync_copy(esrc.at[pl.ds(base, K)], sidx)
            pltpu.sync_copy(edst.at[pl.ds(base, K)], didx)
            pltpu.sync_copy(ew.at[pl.ds(base, K)], wv)
            pltpu.async_copy(chs[_t].at[sidx], rows, sem).wait()

            def _scale(g, c2):
                w16 = wv[pl.ds(g * 16, 16)]
                for ll in range(16):
                    j = g * 16 + ll
                    wsp = w16[ll]
                    for cc in range(8):
                        sl = pl.ds(16 * cc, 16)
                        rows[j, sl] = rows[j, sl] * wsp
                return c2

            lax.fori_loop(0, K // 16, _scale, 0)
            pltpu.sync_copy(rows, acc.at[didx], add=True)
            return carry

        lax.fori_loop(0, nchunks, _chunk, 0)
        plsc.subcore_barrier()

        for b5 in range(5):
            sl = pl.ds(row0 + K * b5, K)
            pltpu.sync_copy(acc.at[sl], outs[t].at[cid, sl])
        plsc.subcore_barrier()


def _full(shape):
    return pl.BlockSpec(shape, lambda i, _s=shape: tuple(0 for _ in _s))


def _rowblk(w):
    return pl.BlockSpec((B, w), lambda i: (i, 0))


def _pre_body(nf, ae, wav, w1, b1, cw, cb, cur_o, c0, c1, c2, c3):
    ws = jnp.sum(wav[...], axis=0, keepdims=True)
    left = jnp.tanh(
        jnp.dot(nf[...], w1[...], preferred_element_type=jnp.float32)
        + b1[...] + ae[...])
    right = jnp.broadcast_to(jnp.tanh(ws), (B, L))
    cur = jnp.concatenate([left, right], axis=1)
    cur_o[...] = cur
    cf = jnp.dot(cur, cw[...], preferred_element_type=jnp.float32) + cb[...]
    c0[...] = cf[:, 0:L]
    c1[...] = cf[:, L:2 * L]
    c2[...] = cf[:, 2 * L:3 * L]
    c3[...] = cf[:, 3 * L:4 * L]


_pre_call = pl.pallas_call(
    _pre_body,
    grid=(GRID,),
    in_specs=[_rowblk(L), _rowblk(L), _full((512, L)), _full((L, L)),
              _full((1, L)), _full((TWO_L, FOUR_L)), _full((1, FOUR_L))],
    out_specs=[_rowblk(TWO_L)] + [_rowblk(L)] * 4,
    out_shape=[jax.ShapeDtypeStruct((N, TWO_L), jnp.float32)]
    + [jax.ShapeDtypeStruct((N, L), jnp.float32)] * 4,
)


def _merge_core(ps, cur, mw, mb):
    mwv = mw[...]
    s = jnp.zeros((B, TWO_L), jnp.float32)
    for i in range(4):
        pv = ps[i][...]
        m = jnp.tanh(pv[0] + pv[1])
        s = s + jnp.dot(m, mwv[i * L:(i + 1) * L, :],
                        preferred_element_type=jnp.float32)
    return jnp.tanh(s + mb[...] + cur[...])


def _merge_conv_body(p0, p1, p2, p3, cur, mw, mb, cw, cb,
                     cur_o, c0, c1, c2, c3):
    cur2 = _merge_core([p0, p1, p2, p3], cur, mw, mb)
    cur_o[...] = cur2
    cf = jnp.dot(cur2, cw[...], preferred_element_type=jnp.float32) + cb[...]
    c0[...] = cf[:, 0:L]
    c1[...] = cf[:, L:2 * L]
    c2[...] = cf[:, 2 * L:3 * L]
    c3[...] = cf[:, 3 * L:4 * L]


def _merge_final_body(p0, p1, p2, p3, cur, mw, mb, cur_o):
    cur_o[...] = _merge_core([p0, p1, p2, p3], cur, mw, mb)


def _pblk():
    return pl.BlockSpec((2, B, L), lambda i: (0, i, 0))


_merge_conv_call = pl.pallas_call(
    _merge_conv_body,
    grid=(GRID,),
    in_specs=[_pblk(), _pblk(), _pblk(), _pblk(),
              _rowblk(TWO_L), _full((FOUR_L, TWO_L)), _full((1, TWO_L)),
              _full((TWO_L, FOUR_L)), _full((1, FOUR_L))],
    out_specs=[_rowblk(TWO_L)] + [_rowblk(L)] * 4,
    out_shape=[jax.ShapeDtypeStruct((N, TWO_L), jnp.float32)]
    + [jax.ShapeDtypeStruct((N, L), jnp.float32)] * 4,
)

_merge_final_call = pl.pallas_call(
    _merge_final_body,
    grid=(GRID,),
    in_specs=[_pblk(), _pblk(), _pblk(), _pblk(),
              _rowblk(TWO_L), _full((FOUR_L, TWO_L)), _full((1, TWO_L))],
    out_specs=_rowblk(TWO_L),
    out_shape=jax.ShapeDtypeStruct((N, TWO_L), jnp.float32),
)


def kernel(node_feat, edge_index, edge_weight, all_embedding, wave_embedding,
           w_n2l_W, w_n2l_b, conv_W, conv_b, merge_W, merge_b):
    cur, c0, c1, c2, c3 = _pre_call(
        node_feat, all_embedding, wave_embedding, w_n2l_W,
        w_n2l_b.reshape(1, L), conv_W[0], conv_b[0].reshape(1, FOUR_L))
    esrc = edge_index[:, 0, :].reshape(-1)
    edst = edge_index[:, 1, :].reshape(-1)
    ew = edge_weight.reshape(-1)
    for lv in range(3):
        p0, p1, p2, p3 = _spmm_all_types(c0, c1, c2, c3, esrc, edst, ew)
        if lv < 2:
            cur, c0, c1, c2, c3 = _merge_conv_call(
                p0, p1, p2, p3, cur, merge_W[lv], merge_b[lv].reshape(1, TWO_L),
                conv_W[lv + 1], conv_b[lv + 1].reshape(1, FOUR_L))
        else:
            cur = _merge_final_call(
                p0, p1, p2, p3, cur, merge_W[2], merge_b[2].reshape(1, TWO_L))
    return cur
